# Initial kernel scaffold; baseline (speedup 1.0000x reference)
import jax
import jax.numpy as jnp
from jax.experimental import pallas as pl


def kernel(x, table):
    raise NotImplementedError("write your pallas kernel here")


# speedup vs baseline: 1.7393x; 1.7393x over previous
import functools

import jax
import jax.numpy as jnp
from jax import lax
from jax.experimental import pallas as pl
from jax.experimental.pallas import tpu as pltpu
from jax.experimental.pallas import tpu_sc as plsc

V_DIM = 1000000
EMB = 32
BATCH = 16384
CTX = 20

NC = 2
NS = 16
NW = NC * NS
BPW = BATCH // NW
IDX_PER_W = BPW * CTX
IDX_CHUNK = 128
ROWS_PER_STEP = 32
GATHERS_PER_STEP = ROWS_PER_STEP * CTX // IDX_CHUNK
N_STEPS = BPW // ROWS_PER_STEP
IDX_ROWS_PER_W = IDX_PER_W // IDX_CHUNK


def _tree_sum(vs):
    while len(vs) > 1:
        nxt = [vs[k] + vs[k + 1] for k in range(0, len(vs) - 1, 2)]
        if len(vs) % 2:
            nxt.append(vs[-1])
        vs = nxt
    return vs[0]


def _cbow_body(x_hbm, tab_hbm, out_hbm, idx_v, buf0, buf1, out_v, sem0, sem1):
    wid = lax.axis_index("s") * NC + lax.axis_index("c")

    pltpu.sync_copy(x_hbm.at[pl.ds(wid * IDX_ROWS_PER_W, IDX_ROWS_PER_W)], idx_v)

    bufs = (buf0, buf1)
    sems = (sem0, sem1)

    def fire(step, slot):
        cps = []
        for j in range(GATHERS_PER_STEP):
            cps.append(
                pltpu.async_copy(
                    tab_hbm.at[idx_v.at[step * GATHERS_PER_STEP + j]],
                    bufs[slot].at[pl.ds(j * IDX_CHUNK, IDX_CHUNK)],
                    sems[slot],
                )
            )
        return cps

    def reduce_step(step, slot):
        buf = bufs[slot]
        inv = jnp.float32(1.0 / CTX)

        def row_body(i, carry):
            base = i * CTX
            lo = [buf[base + j, 0:16] for j in range(CTX)]
            hi = [buf[base + j, 16:32] for j in range(CTX)]
            o = step * ROWS_PER_STEP + i
            out_v[o, 0:16] = _tree_sum(lo) * inv
            out_v[o, 16:32] = _tree_sum(hi) * inv
            return carry

        lax.fori_loop(0, ROWS_PER_STEP, row_body, 0)

    inflight = [fire(0, 0), fire(1, 1)]
    for g in range(N_STEPS):
        slot = g % 2
        for cp in inflight[slot]:
            cp.wait()
        reduce_step(g, slot)
        if g + 2 < N_STEPS:
            inflight[slot] = fire(g + 2, slot)

    pltpu.sync_copy(out_v, out_hbm.at[pl.ds(wid * BPW, BPW)])


@jax.jit
def _cbow(x2d, table):
    mesh = plsc.VectorSubcoreMesh(core_axis_name="c", subcore_axis_name="s")
    return pl.kernel(
        _cbow_body,
        out_type=jax.ShapeDtypeStruct((BATCH, EMB), jnp.float32),
        mesh=mesh,
        compiler_params=pltpu.CompilerParams(use_tc_tiling_on_sc=False),
        scratch_types=[
            pltpu.VMEM((IDX_ROWS_PER_W, IDX_CHUNK), jnp.int32),
            pltpu.VMEM((ROWS_PER_STEP * CTX, EMB), jnp.float32),
            pltpu.VMEM((ROWS_PER_STEP * CTX, EMB), jnp.float32),
            pltpu.VMEM((BPW, EMB), jnp.float32),
            pltpu.SemaphoreType.DMA,
            pltpu.SemaphoreType.DMA,
        ],
    )(x2d, table)


def kernel(x, table):
    x2d = x.astype(jnp.int32).reshape(BATCH * CTX // IDX_CHUNK, IDX_CHUNK)
    return _cbow(x2d, table)
